# baseline (speedup 1.0000x reference)
import functools

import jax
import jax.numpy as jnp
from jax import lax
from jax.experimental import pallas as pl
from jax.experimental.pallas import tpu as pltpu
from jax.experimental.pallas import tpu_sc as plsc

VOCAB = 1000000
EMBED_DIM = 64
BATCH = 16384
HIST = 200

_INFO = plsc.get_sparse_core_info()
NC, NS = _INFO.num_cores, _INFO.num_subcores
NW = NC * NS

B_TOTAL = BATCH * HIST
B_PER_W = B_TOTAL // NW
CHUNK = 128
NCHUNKS = B_PER_W // CHUNK
NSLOT = 4
CPB = 8
CPJ = 2 * CPB
NBODY = NCHUNKS // CPJ
IDS2_COLS = 128
IDS2_ROWS = B_TOTAL // IDS2_COLS
IDROWS_PER_W = B_PER_W // IDS2_COLS

assert B_PER_W * NW == B_TOTAL
assert NCHUNKS * CHUNK == B_PER_W
assert NBODY * CPJ == NCHUNKS


@functools.partial(
    pl.kernel,
    mesh=plsc.VectorSubcoreMesh(core_axis_name="c", subcore_axis_name="s"),
    out_type=jax.ShapeDtypeStruct((B_TOTAL, 128), jnp.float32),
    scratch_types=[
        pltpu.VMEM((CPB, 128), jnp.int32),
        pltpu.VMEM((CPB, 128), jnp.int32),
        pltpu.VMEM((CHUNK, 128), jnp.float32),
        pltpu.VMEM((CHUNK, 128), jnp.float32),
        pltpu.VMEM((CHUNK, 128), jnp.float32),
        pltpu.VMEM((CHUNK, 128), jnp.float32),
        pltpu.SemaphoreType.DMA,
        pltpu.SemaphoreType.DMA,
        pltpu.SemaphoreType.DMA,
        pltpu.SemaphoreType.DMA,
        pltpu.SemaphoreType.DMA,
        pltpu.SemaphoreType.DMA,
        pltpu.SemaphoreType.DMA,
        pltpu.SemaphoreType.DMA,
        pltpu.SemaphoreType.DMA,
        pltpu.SemaphoreType.DMA,
    ],
)
def _gather_kernel(ids2_hbm, t128_hbm, out_hbm, ib0, ib1,
                   r0, r1, r2, r3, sib0, sib1,
                   sg0, sg1, sg2, sg3, ss0, ss1, ss2, ss3):
    wid = lax.axis_index("s") * NC + lax.axis_index("c")
    wflat = wid * B_PER_W
    widrow = wid * IDROWS_PER_W

    ib = (ib0, ib1)
    rows = (r0, r1, r2, r3)
    sib = (sib0, sib1)
    sg = (sg0, sg1, sg2, sg3)
    ss = (ss0, ss1, ss2, ss3)

    def idx_block_copy(blk, buf):
        return pltpu.make_async_copy(
            ids2_hbm.at[pl.ds(widrow + blk * CPB, CPB)], ib[buf], sib[buf])

    def gather_copy(k, s, buf, row):
        return pltpu.make_async_copy(
            t128_hbm.at[ib[buf].at[row]], rows[s], sg[s])

    def store_copy(k, s):
        return pltpu.make_async_copy(
            rows[s], out_hbm.at[pl.ds(wflat + k * CHUNK, CHUNK)], ss[s])

    def chunk_step(jj, i):
        k = jj * CPJ + i
        s = i % NSLOT
        buf = (i // CPB) % 2

        def wait_reuse():
            store_copy(k - NSLOT, s).wait()

        def finish_prev():
            pk = k - 2
            pi = i - 2
            ps = pi % NSLOT
            pbuf = ((pi % CPJ) // CPB) % 2
            prow = pi % CPB
            gather_copy(pk, ps, pbuf, prow).wait()
            store_copy(pk, ps).start()

        if i < NSLOT:
            pl.when(jj >= 1)(wait_reuse)
        else:
            wait_reuse()
        gather_copy(k, s, buf, i % CPB).start()
        if i < 2:
            pl.when(jj >= 1)(finish_prev)
        else:
            finish_prev()

    def body(jj, carry):
        idx_block_copy(2 * jj, 0).wait()
        for i in range(CPB):
            chunk_step(jj, i)
            if i == 1:
                idx_block_copy(2 * jj + 1, 1).start()
        idx_block_copy(2 * jj + 1, 1).wait()
        for i in range(CPB, CPJ):
            chunk_step(jj, i)
            if i == CPB + 1:
                pl.when(jj < NBODY - 1)(
                    lambda: idx_block_copy(2 * jj + 2, 0).start())
        return carry

    idx_block_copy(0, 0).start()
    lax.fori_loop(0, NBODY, body, 0)

    last = NCHUNKS - 1
    for k in (last - 1, last):
        i = k % CPJ
        gather_copy(k, i % NSLOT, (i // CPB) % 2, i % CPB).wait()
        store_copy(k, i % NSLOT).start()
    for k in range(last - 3, last + 1):
        store_copy(k, (k % CPJ) % NSLOT).wait()


_PAD_ROWS = 8000


def _pad_body(x_ref, o_ref):
    x = x_ref[...]
    o_ref[...] = jnp.concatenate([x, jnp.zeros_like(x)], axis=1)


_pad_table = pl.pallas_call(
    _pad_body,
    grid=(VOCAB // _PAD_ROWS,),
    in_specs=[pl.BlockSpec((_PAD_ROWS, EMBED_DIM), lambda i: (i, 0))],
    out_specs=pl.BlockSpec((_PAD_ROWS, 128), lambda i: (i, 0)),
    out_shape=jax.ShapeDtypeStruct((VOCAB, 128), jnp.float32),
)


def kernel(input_ids, table):
    ids2 = input_ids.reshape(IDS2_ROWS, IDS2_COLS)
    t128 = _pad_table(table)
    out128 = _gather_kernel(ids2, t128)
    return lax.slice(out128, (0, 0), (B_TOTAL, EMBED_DIM)).reshape(
        BATCH, HIST, EMBED_DIM)


# speedup vs baseline: 1.0591x; 1.0591x over previous
import functools

import jax
import jax.numpy as jnp
from jax import lax
from jax.experimental import pallas as pl
from jax.experimental.pallas import tpu as pltpu
from jax.experimental.pallas import tpu_sc as plsc

VOCAB = 1000000
EMBED_DIM = 64
BATCH = 16384
HIST = 200

_INFO = plsc.get_sparse_core_info()
NC, NS = _INFO.num_cores, _INFO.num_subcores
NW = NC * NS

B_TOTAL = BATCH * HIST
B_PER_W = B_TOTAL // NW
CHUNK = 128
NCHUNKS = B_PER_W // CHUNK
NSLOT = 4
CPB = 8
CPJ = 2 * CPB
NBODY = NCHUNKS // CPJ
IDS2_COLS = 128
IDS2_ROWS = B_TOTAL // IDS2_COLS
IDROWS_PER_W = B_PER_W // IDS2_COLS

assert B_PER_W * NW == B_TOTAL
assert NCHUNKS * CHUNK == B_PER_W
assert NBODY * CPJ == NCHUNKS


@functools.partial(
    pl.kernel,
    mesh=plsc.VectorSubcoreMesh(core_axis_name="c", subcore_axis_name="s"),
    out_type=jax.ShapeDtypeStruct((B_TOTAL, 128), jnp.float32),
    scratch_types=[
        pltpu.VMEM((CPB, 128), jnp.int32),
        pltpu.VMEM((CPB, 128), jnp.int32),
        pltpu.VMEM((CHUNK, 128), jnp.float32),
        pltpu.VMEM((CHUNK, 128), jnp.float32),
        pltpu.VMEM((CHUNK, 128), jnp.float32),
        pltpu.VMEM((CHUNK, 128), jnp.float32),
        pltpu.SemaphoreType.DMA,
        pltpu.SemaphoreType.DMA,
        pltpu.SemaphoreType.DMA,
        pltpu.SemaphoreType.DMA,
        pltpu.SemaphoreType.DMA,
        pltpu.SemaphoreType.DMA,
        pltpu.SemaphoreType.DMA,
        pltpu.SemaphoreType.DMA,
        pltpu.SemaphoreType.DMA,
        pltpu.SemaphoreType.DMA,
    ],
)
def _gather_kernel(ids2_hbm, t128_hbm, out_hbm, ib0, ib1,
                   r0, r1, r2, r3, sib0, sib1,
                   sg0, sg1, sg2, sg3, ss0, ss1, ss2, ss3):
    wid = lax.axis_index("s") * NC + lax.axis_index("c")
    wflat = wid * B_PER_W
    widrow = wid * IDROWS_PER_W

    ib = (ib0, ib1)
    rows = (r0, r1, r2, r3)
    sib = (sib0, sib1)
    sg = (sg0, sg1, sg2, sg3)
    ss = (ss0, ss1, ss2, ss3)

    def idx_block_copy(blk, buf):
        return pltpu.make_async_copy(
            ids2_hbm.at[pl.ds(widrow + blk * CPB, CPB)], ib[buf], sib[buf])

    def gather_copy(k, s, buf, row):
        return pltpu.make_async_copy(
            t128_hbm.at[ib[buf].at[row]], rows[s], sg[s])

    def store_copy(k, s):
        return pltpu.make_async_copy(
            rows[s], out_hbm.at[pl.ds(wflat + k * CHUNK, CHUNK)], ss[s])

    def chunk_step(jj, i):
        k = jj * CPJ + i
        s = i % NSLOT
        buf = (i // CPB) % 2

        def wait_reuse():
            store_copy(k - NSLOT, s).wait()

        def finish_prev():
            pk = k - 2
            pi = i - 2
            ps = pi % NSLOT
            pbuf = ((pi % CPJ) // CPB) % 2
            prow = pi % CPB
            gather_copy(pk, ps, pbuf, prow).wait()
            store_copy(pk, ps).start()

        if i < NSLOT:
            pl.when(jj >= 1)(wait_reuse)
        else:
            wait_reuse()
        gather_copy(k, s, buf, i % CPB).start()
        if i < 2:
            pl.when(jj >= 1)(finish_prev)
        else:
            finish_prev()

    def body(jj, carry):
        idx_block_copy(2 * jj, 0).wait()
        for i in range(CPB):
            chunk_step(jj, i)
            if i == 1:
                idx_block_copy(2 * jj + 1, 1).start()
        idx_block_copy(2 * jj + 1, 1).wait()
        for i in range(CPB, CPJ):
            chunk_step(jj, i)
            if i == CPB + 1:
                pl.when(jj < NBODY - 1)(
                    lambda: idx_block_copy(2 * jj + 2, 0).start())
        return carry

    idx_block_copy(0, 0).start()
    lax.fori_loop(0, NBODY, body, 0)

    last = NCHUNKS - 1
    for k in (last - 1, last):
        i = k % CPJ
        gather_copy(k, i % NSLOT, (i // CPB) % 2, i % CPB).wait()
        store_copy(k, i % NSLOT).start()
    for k in range(last - 3, last + 1):
        store_copy(k, (k % CPJ) % NSLOT).wait()


def kernel(input_ids, table):
    ids2 = input_ids.reshape(IDS2_ROWS, IDS2_COLS)
    t128 = lax.dynamic_update_slice(
        jnp.zeros((VOCAB, 128), jnp.float32), table, (0, 0))
    out128 = _gather_kernel(ids2, t128)
    return lax.slice(out128, (0, 0), (B_TOTAL, EMBED_DIM)).reshape(
        BATCH, HIST, EMBED_DIM)
